# baseline (speedup 1.0000x reference)
import functools

import jax
import jax.numpy as jnp
from jax import lax
from jax.experimental import pallas as pl
from jax.experimental.pallas import tpu as pltpu

N = 10000
E = 160000
L = 256
NBLK = 1000
EBLK = 2000
NB = N // NBLK
EB = E // EBLK
F32 = jnp.float32


def _dot(a, b):
    return jax.lax.dot_general(a, b, (((1,), (0,)), ((), ())),
                               preferred_element_type=F32)


def _prep_body(ee_w2, ee_b2, w1e0, w1e1, emb1_0, emb1_1, emw2_0, emw2_1,
               nmw1a0, nmw1a1, emb2_0, emb2_1,
               We0, We1, bb0, bb1, Wf0, Wf1, vf0, vf1):
    We0[...] = _dot(ee_w2[...], w1e0[...])
    We1[...] = _dot(ee_w2[...], w1e1[...])
    bb0[...] = emb1_0[...] + _dot(ee_b2[...], w1e0[...])
    bb1[...] = emb1_1[...] + _dot(ee_b2[...], w1e1[...])
    Wf0[...] = _dot(emw2_0[...], nmw1a0[...])
    Wf1[...] = _dot(emw2_1[...], nmw1a1[...])
    vf0[...] = _dot(emb2_0[...], nmw1a0[...])
    vf1[...] = _dot(emb2_1[...], nmw1a1[...])


def _prep(p):
    mat = jax.ShapeDtypeStruct((L, L), F32)
    vec = jax.ShapeDtypeStruct((1, L), F32)
    return pl.pallas_call(
        _prep_body,
        out_shape=[mat, mat, vec, vec, mat, mat, vec, vec],
    )(p['ee_w2'], p['ee_b2'][None, :],
      p['l0_em_w1'][2 * L:], p['l1_em_w1'][2 * L:],
      p['l0_em_b1'][None, :], p['l1_em_b1'][None, :],
      p['l0_em_w2'], p['l1_em_w2'],
      p['l0_nm_w1'][L:], p['l1_nm_w1'][L:],
      p['l0_em_b2'][None, :], p['l1_em_b2'][None, :])


def _enc_body(x, w1, b1, w2, b2, h):
    t = jnp.maximum(_dot(x[...], w1[...]) + b1[...], 0.0)
    h[...] = _dot(t, w2[...]) + b2[...]


def _node_encode(x, p):
    return pl.pallas_call(
        _enc_body,
        grid=(NB,),
        in_specs=[
            pl.BlockSpec((NBLK, 128), lambda i: (i, 0)),
            pl.BlockSpec((128, L), lambda i: (0, 0)),
            pl.BlockSpec((1, L), lambda i: (0, 0)),
            pl.BlockSpec((L, L), lambda i: (0, 0)),
            pl.BlockSpec((1, L), lambda i: (0, 0)),
        ],
        out_specs=pl.BlockSpec((NBLK, L), lambda i: (i, 0)),
        out_shape=jax.ShapeDtypeStruct((N, L), F32),
    )(x, p['ne_w1'], p['ne_b1'][None, :], p['ne_w2'], p['ne_b2'][None, :])


def _edge_body(ea, w1, b1, We0, We1, C0, C1):
    g = jnp.maximum(_dot(ea[...], w1[...]) + b1[...], 0.0)
    C0[...] = _dot(g, We0[...])
    C1[...] = _dot(g, We1[...])


def _edge_C(edge_attr, p, We0, We1):
    cshape = jax.ShapeDtypeStruct((2 * E, 128), F32)
    return pl.pallas_call(
        _edge_body,
        grid=(2, EB),
        in_specs=[
            pl.BlockSpec((EBLK, 16), lambda h, i: (i, 0)),
            pl.BlockSpec((16, L), lambda h, i: (0, 0)),
            pl.BlockSpec((1, L), lambda h, i: (0, 0)),
            pl.BlockSpec((L, 128), lambda h, i: (0, h)),
            pl.BlockSpec((L, 128), lambda h, i: (0, h)),
        ],
        out_specs=[
            pl.BlockSpec((EBLK, 128), lambda h, i: (h * EB + i, 0)),
            pl.BlockSpec((EBLK, 128), lambda h, i: (h * EB + i, 0)),
        ],
        out_shape=[cshape, cshape],
    )(edge_attr, p['ee_w1'], p['ee_b1'][None, :], We0, We1)


def _ab_body(h, w1d, w1s, bb, A, B):
    hv = h[...]
    A[...] = _dot(hv, w1d[...]) + bb[...]
    B[...] = _dot(hv, w1s[...])


def _ab(h, w1d, w1s, bb):
    oshape = jax.ShapeDtypeStruct((2 * N, 128), F32)
    return pl.pallas_call(
        _ab_body,
        grid=(2, NB),
        in_specs=[
            pl.BlockSpec((NBLK, L), lambda hf, i: (i, 0)),
            pl.BlockSpec((L, 128), lambda hf, i: (0, hf)),
            pl.BlockSpec((L, 128), lambda hf, i: (0, hf)),
            pl.BlockSpec((1, 128), lambda hf, i: (0, hf)),
        ],
        out_specs=[
            pl.BlockSpec((NBLK, 128), lambda hf, i: (hf * NB + i, 0)),
            pl.BlockSpec((NBLK, 128), lambda hf, i: (hf * NB + i, 0)),
        ],
        out_shape=[oshape, oshape],
    )(h, w1d, w1s, bb)


def _upd_body(h, St, Sb, dg, W1h, Wf, b1, vf, w2, b2, hn):
    hv = h[...]
    wf = Wf[...]
    t = (_dot(hv, W1h[...]) + _dot(St[...], wf[:128]) +
         _dot(Sb[...], wf[128:]) + dg[:, 0:1] * vf[...] + b1[...])
    t = jnp.maximum(t, 0.0)
    hn[...] = hv + _dot(t, w2[...]) + b2[...]


def _node_update(h, S2, deg2d, W1h, Wf, b1, vf, w2, b2):
    return pl.pallas_call(
        _upd_body,
        grid=(NB,),
        in_specs=[
            pl.BlockSpec((NBLK, L), lambda i: (i, 0)),
            pl.BlockSpec((NBLK, 128), lambda i: (i, 0)),
            pl.BlockSpec((NBLK, 128), lambda i: (NB + i, 0)),
            pl.BlockSpec((NBLK, 16), lambda i: (i, 0)),
            pl.BlockSpec((L, L), lambda i: (0, 0)),
            pl.BlockSpec((L, L), lambda i: (0, 0)),
            pl.BlockSpec((1, L), lambda i: (0, 0)),
            pl.BlockSpec((1, L), lambda i: (0, 0)),
            pl.BlockSpec((L, L), lambda i: (0, 0)),
            pl.BlockSpec((1, L), lambda i: (0, 0)),
        ],
        out_specs=pl.BlockSpec((NBLK, L), lambda i: (i, 0)),
        out_shape=jax.ShapeDtypeStruct((N, L), F32),
    )(h, S2, S2, deg2d, W1h, Wf, b1, vf, w2, b2)


def _dec_body(h, w1, b1, w2, b2, o):
    t = jnp.maximum(_dot(h[...], w1[...]) + b1[...], 0.0)
    o[...] = _dot(t, w2[...]) + b2[...]


def _decode(h, p):
    w2p = jnp.pad(p['de_w2'], ((0, 0), (0, 128 - 3)))
    b2p = jnp.pad(p['de_b2'], ((0, 128 - 3)))[None, :]
    out = pl.pallas_call(
        _dec_body,
        grid=(NB,),
        in_specs=[
            pl.BlockSpec((NBLK, L), lambda i: (i, 0)),
            pl.BlockSpec((L, L), lambda i: (0, 0)),
            pl.BlockSpec((1, L), lambda i: (0, 0)),
            pl.BlockSpec((L, 128), lambda i: (0, 0)),
            pl.BlockSpec((1, 128), lambda i: (0, 0)),
        ],
        out_specs=pl.BlockSpec((NBLK, 128), lambda i: (i, 0)),
        out_shape=jax.ShapeDtypeStruct((N, 128), F32),
    )(h, p['de_w1'], p['de_b1'][None, :], w2p, b2p)
    return out[:, :3]


def _segment_pass(A2, B2, C2, dst, src, need_deg):
    Af = jnp.concatenate([A2[:N], A2[N:]], axis=1)
    Bf = jnp.concatenate([B2[:N], B2[N:]], axis=1)
    Cf = jnp.concatenate([C2[:E], C2[E:]], axis=1)
    t = jnp.maximum(Af[dst] + Bf[src] + Cf, 0.0)
    S = jax.ops.segment_sum(t, dst, num_segments=N)
    S2 = jnp.concatenate([S[:, :128], S[:, 128:]], axis=0)
    deg2d = None
    if need_deg:
        deg = jax.ops.segment_sum(jnp.ones((E,), F32), dst, num_segments=N)
        deg2d = jnp.tile(deg[:, None], (1, 16))
    return S2, deg2d


def kernel(x, edge_index, edge_attr, params):
    p = params
    src = edge_index[0]
    dst = edge_index[1]

    We0, We1, bb0, bb1, Wf0, Wf1, vf0, vf1 = _prep(p)
    h = _node_encode(x, p)
    C0, C1 = _edge_C(edge_attr, p, We0, We1)

    deg2d = None
    for i, (We, bb, Wf, vf, C) in enumerate(
            [(We0, bb0, Wf0, vf0, C0), (We1, bb1, Wf1, vf1, C1)]):
        w1 = p[f'l{i}_em_w1']
        A2, B2 = _ab(h, w1[:L], w1[L:2 * L], bb)
        S2, dg = _segment_pass(A2, B2, C, dst, src, need_deg=(i == 0))
        if dg is not None:
            deg2d = dg
        h = _node_update(h, S2, deg2d, p[f'l{i}_nm_w1'][:L], Wf,
                         p[f'l{i}_nm_b1'][None, :], vf,
                         p[f'l{i}_nm_w2'], p[f'l{i}_nm_b2'][None, :])

    return _decode(h, p)


# speedup vs baseline: 2.3393x; 2.3393x over previous
import functools

import jax
import jax.numpy as jnp
from jax import lax
from jax.experimental import pallas as pl
from jax.experimental.pallas import tpu as pltpu
from jax.experimental.pallas import tpu_sc as plsc

N = 10000
E = 160000
L = 256
NBLK = 1000
EBLK = 2000
NB = N // NBLK
EB = E // EBLK
F32 = jnp.float32


def _dot(a, b):
    return jax.lax.dot_general(a, b, (((1,), (0,)), ((), ())),
                               preferred_element_type=F32)


def _prep_body(ee_w2, ee_b2, w1e0, w1e1, emb1_0, emb1_1, emw2_0, emw2_1,
               nmw1a0, nmw1a1, emb2_0, emb2_1,
               We0, We1, bb0, bb1, Wf0, Wf1, vf0, vf1):
    We0[...] = _dot(ee_w2[...], w1e0[...])
    We1[...] = _dot(ee_w2[...], w1e1[...])
    bb0[...] = emb1_0[...] + _dot(ee_b2[...], w1e0[...])
    bb1[...] = emb1_1[...] + _dot(ee_b2[...], w1e1[...])
    Wf0[...] = _dot(emw2_0[...], nmw1a0[...])
    Wf1[...] = _dot(emw2_1[...], nmw1a1[...])
    vf0[...] = _dot(emb2_0[...], nmw1a0[...])
    vf1[...] = _dot(emb2_1[...], nmw1a1[...])


def _prep(p):
    mat = jax.ShapeDtypeStruct((L, L), F32)
    vec = jax.ShapeDtypeStruct((1, L), F32)
    return pl.pallas_call(
        _prep_body,
        out_shape=[mat, mat, vec, vec, mat, mat, vec, vec],
    )(p['ee_w2'], p['ee_b2'][None, :],
      p['l0_em_w1'][2 * L:], p['l1_em_w1'][2 * L:],
      p['l0_em_b1'][None, :], p['l1_em_b1'][None, :],
      p['l0_em_w2'], p['l1_em_w2'],
      p['l0_nm_w1'][L:], p['l1_nm_w1'][L:],
      p['l0_em_b2'][None, :], p['l1_em_b2'][None, :])


def _enc_body(x, w1, b1, w2, b2, h):
    t = jnp.maximum(_dot(x[...], w1[...]) + b1[...], 0.0)
    h[...] = _dot(t, w2[...]) + b2[...]


def _node_encode(x, p):
    return pl.pallas_call(
        _enc_body,
        grid=(NB,),
        in_specs=[
            pl.BlockSpec((NBLK, 128), lambda i: (i, 0)),
            pl.BlockSpec((128, L), lambda i: (0, 0)),
            pl.BlockSpec((1, L), lambda i: (0, 0)),
            pl.BlockSpec((L, L), lambda i: (0, 0)),
            pl.BlockSpec((1, L), lambda i: (0, 0)),
        ],
        out_specs=pl.BlockSpec((NBLK, L), lambda i: (i, 0)),
        out_shape=jax.ShapeDtypeStruct((N, L), F32),
    )(x, p['ne_w1'], p['ne_b1'][None, :], p['ne_w2'], p['ne_b2'][None, :])


def _edge_body(ea, w1, b1, We0, We1, C0, C1):
    g = jnp.maximum(_dot(ea[...], w1[...]) + b1[...], 0.0)
    C0[...] = _dot(g, We0[...])
    C1[...] = _dot(g, We1[...])


def _edge_C(edge_attr, p, We0, We1):
    cshape = jax.ShapeDtypeStruct((2 * E, 128), F32)
    return pl.pallas_call(
        _edge_body,
        grid=(2, EB),
        in_specs=[
            pl.BlockSpec((EBLK, 16), lambda h, i: (i, 0)),
            pl.BlockSpec((16, L), lambda h, i: (0, 0)),
            pl.BlockSpec((1, L), lambda h, i: (0, 0)),
            pl.BlockSpec((L, 128), lambda h, i: (0, h)),
            pl.BlockSpec((L, 128), lambda h, i: (0, h)),
        ],
        out_specs=[
            pl.BlockSpec((EBLK, 128), lambda h, i: (h * EB + i, 0)),
            pl.BlockSpec((EBLK, 128), lambda h, i: (h * EB + i, 0)),
        ],
        out_shape=[cshape, cshape],
    )(edge_attr, p['ee_w1'], p['ee_b1'][None, :], We0, We1)


def _ab_body(h, w1d, w1s, bb, A0, A1, B0, B1):
    hv = h[...]
    a = _dot(hv, w1d[...]) + bb[...]
    b = _dot(hv, w1s[...])
    A0[...] = a[:, :128]
    A1[...] = a[:, 128:]
    B0[...] = b[:, :128]
    B1[...] = b[:, 128:]


def _ab(h, w1d, w1s, bb):
    oshape = jax.ShapeDtypeStruct((N, 128), F32)
    return pl.pallas_call(
        _ab_body,
        grid=(NB,),
        in_specs=[
            pl.BlockSpec((NBLK, L), lambda i: (i, 0)),
            pl.BlockSpec((L, L), lambda i: (0, 0)),
            pl.BlockSpec((L, L), lambda i: (0, 0)),
            pl.BlockSpec((1, L), lambda i: (0, 0)),
        ],
        out_specs=[
            pl.BlockSpec((NBLK, 128), lambda i: (i, 0)),
            pl.BlockSpec((NBLK, 128), lambda i: (i, 0)),
            pl.BlockSpec((NBLK, 128), lambda i: (i, 0)),
            pl.BlockSpec((NBLK, 128), lambda i: (i, 0)),
        ],
        out_shape=[oshape, oshape, oshape, oshape],
    )(h, w1d, w1s, bb)


def _upd_body(h, St, Sb, dg, W1h, Wf, b1, vf, w2, b2, hn):
    hv = h[...]
    wf = Wf[...]
    t = (_dot(hv, W1h[...]) + _dot(St[...], wf[:128]) +
         _dot(Sb[...], wf[128:]) + dg[:, 0:1] * vf[...] + b1[...])
    t = jnp.maximum(t, 0.0)
    hn[...] = hv + _dot(t, w2[...]) + b2[...]


def _node_update(h, S2, deg2d, W1h, Wf, b1, vf, w2, b2):
    return pl.pallas_call(
        _upd_body,
        grid=(NB,),
        in_specs=[
            pl.BlockSpec((NBLK, L), lambda i: (i, 0)),
            pl.BlockSpec((NBLK, 128), lambda i: (i, 0)),
            pl.BlockSpec((NBLK, 128), lambda i: (NB + i, 0)),
            pl.BlockSpec((NBLK, 128), lambda i: (i, 0)),
            pl.BlockSpec((L, L), lambda i: (0, 0)),
            pl.BlockSpec((L, L), lambda i: (0, 0)),
            pl.BlockSpec((1, L), lambda i: (0, 0)),
            pl.BlockSpec((1, L), lambda i: (0, 0)),
            pl.BlockSpec((L, L), lambda i: (0, 0)),
            pl.BlockSpec((1, L), lambda i: (0, 0)),
        ],
        out_specs=pl.BlockSpec((NBLK, L), lambda i: (i, 0)),
        out_shape=jax.ShapeDtypeStruct((N, L), F32),
    )(h, S2, S2, deg2d, W1h, Wf, b1, vf, w2, b2)


def _dec_body(h, w1, b1, w2, b2, o):
    t = jnp.maximum(_dot(h[...], w1[...]) + b1[...], 0.0)
    o[...] = _dot(t, w2[...]) + b2[...]


def _decode(h, p):
    w2p = jnp.pad(p['de_w2'], ((0, 0), (0, 128 - 3)))
    b2p = jnp.pad(p['de_b2'], ((0, 128 - 3)))[None, :]
    out = pl.pallas_call(
        _dec_body,
        grid=(NB,),
        in_specs=[
            pl.BlockSpec((NBLK, L), lambda i: (i, 0)),
            pl.BlockSpec((L, L), lambda i: (0, 0)),
            pl.BlockSpec((1, L), lambda i: (0, 0)),
            pl.BlockSpec((L, 128), lambda i: (0, 0)),
            pl.BlockSpec((1, 128), lambda i: (0, 0)),
        ],
        out_specs=pl.BlockSpec((NBLK, 128), lambda i: (i, 0)),
        out_shape=jax.ShapeDtypeStruct((N, 128), F32),
    )(h, p['de_w1'], p['de_b1'][None, :], w2p, b2p)
    return out[:, :3]


CH = 80
EPW = E // 16
NCHK = EPW // CH
NSLAB = 624
NTAIL = N - 16 * NSLAB


def _slabbed(copy_chunk, s):
    for j in range(NSLAB // CH):
        copy_chunk(s * NSLAB + j * CH, CH)
    rem = NSLAB % CH
    if rem:
        copy_chunk(s * NSLAB + (NSLAB // CH) * CH, rem)

    @pl.when(s == 15)
    def _():
        copy_chunk(16 * NSLAB, NTAIL)


def _sc_body(dst, src, A0, A1, B0, B1, C2, S2, s_sh,
             idx_d, idx_s, abuf, bbuf, cbuf, sem_a, sem_b):
    c = lax.axis_index("c")
    s = lax.axis_index("s")
    cN = c * N

    zero = jnp.zeros((16,), F32)

    def _init_row(r, carry):
        for k2 in range(8):
            abuf[r, pl.ds(k2 * 16, 16)] = zero
        return carry
    lax.fori_loop(0, CH, _init_row, 0)

    _slabbed(lambda o, n: pltpu.sync_copy(
        abuf.at[pl.ds(0, n)], s_sh.at[pl.ds(o, n)]), s)
    plsc.subcore_barrier()

    def _chunk(j, carry):
        base = s * EPW + j * CH
        pltpu.sync_copy(dst.at[pl.ds(base, CH)], idx_d)
        pltpu.sync_copy(src.at[pl.ds(base, CH)], idx_s)
        pltpu.sync_copy(C2.at[pl.ds(c * E + base, CH)], cbuf)

        def _gathers(At, Bt):
            cp_a = pltpu.async_copy(At.at[idx_d], abuf, sem_a)
            cp_b = pltpu.async_copy(Bt.at[idx_s], bbuf, sem_b)
            cp_a.wait()
            cp_b.wait()

        @pl.when(c == 0)
        def _():
            _gathers(A0, B0)

        @pl.when(c == 1)
        def _():
            _gathers(A1, B1)

        def _row(r, rcarry):
            for k2 in range(8):
                sl = pl.ds(k2 * 16, 16)
                abuf[r, sl] = jnp.maximum(
                    abuf[r, sl] + bbuf[r, sl] + cbuf[r, sl], 0.0)
            return rcarry
        lax.fori_loop(0, CH, _row, 0)

        pltpu.sync_copy(abuf, s_sh.at[idx_d], add=True)
        return carry

    lax.fori_loop(0, NCHK, _chunk, 0)
    plsc.subcore_barrier()

    def _wb_s(o, n):
        pltpu.sync_copy(s_sh.at[pl.ds(o, n)], abuf.at[pl.ds(0, n)])
        pltpu.sync_copy(abuf.at[pl.ds(0, n)], S2.at[pl.ds(cN + o, n)])
    _slabbed(_wb_s, s)


def _make_sc():
    i32 = jnp.int32
    return pl.kernel(
        _sc_body,
        out_type=[jax.ShapeDtypeStruct((2 * N, 128), F32)],
        mesh=plsc.VectorSubcoreMesh(core_axis_name="c", subcore_axis_name="s"),
        scratch_types=[
            pltpu.VMEM_SHARED((N, 128), F32),
            pltpu.VMEM((CH,), i32), pltpu.VMEM((CH,), i32),
            pltpu.VMEM((CH, 128), F32), pltpu.VMEM((CH, 128), F32),
            pltpu.VMEM((CH, 128), F32),
            pltpu.SemaphoreType.DMA, pltpu.SemaphoreType.DMA,
        ],
    )


def _deg_body(dst, deg2d, d_sh, idx_d, obuf):
    c = lax.axis_index("c")
    s = lax.axis_index("s")

    zero = jnp.zeros((16,), F32)
    one = jnp.full((16,), 1.0, dtype=F32)

    def _fill(val):
        def _row(r, carry):
            for k2 in range(8):
                obuf[r, pl.ds(k2 * 16, 16)] = val
            return carry
        lax.fori_loop(0, CH, _row, 0)

    _fill(zero)

    @pl.when(c == 0)
    def _():
        _slabbed(lambda o, n: pltpu.sync_copy(
            obuf.at[pl.ds(0, n)], d_sh.at[pl.ds(o, n)]), s)
    _fill(one)
    plsc.subcore_barrier()

    @pl.when(c == 0)
    def _():
        def _chunk(j, carry):
            base = s * EPW + j * CH
            pltpu.sync_copy(dst.at[pl.ds(base, CH)], idx_d)
            pltpu.sync_copy(obuf, d_sh.at[idx_d], add=True)
            return carry
        lax.fori_loop(0, NCHK, _chunk, 0)
    plsc.subcore_barrier()

    @pl.when(c == 0)
    def _():
        def _wb_d(o, n):
            pltpu.sync_copy(d_sh.at[pl.ds(o, n)], obuf.at[pl.ds(0, n)])
            pltpu.sync_copy(obuf.at[pl.ds(0, n)], deg2d.at[pl.ds(o, n)])
        _slabbed(_wb_d, s)


def _make_deg():
    return pl.kernel(
        _deg_body,
        out_type=[jax.ShapeDtypeStruct((N, 128), F32)],
        mesh=plsc.VectorSubcoreMesh(core_axis_name="c", subcore_axis_name="s"),
        scratch_types=[
            pltpu.VMEM_SHARED((N, 128), F32),
            pltpu.VMEM((CH,), jnp.int32),
            pltpu.VMEM((CH, 128), F32),
        ],
    )


def _segment_pass(AB, C2, dst, src):
    A0, A1, B0, B1 = AB
    (S2,) = _make_sc()(dst, src, A0, A1, B0, B1, C2)
    return S2


def kernel(x, edge_index, edge_attr, params):
    p = params
    src = edge_index[0]
    dst = edge_index[1]

    We0, We1, bb0, bb1, Wf0, Wf1, vf0, vf1 = _prep(p)
    h = _node_encode(x, p)
    C0, C1 = _edge_C(edge_attr, p, We0, We1)

    (deg2d,) = _make_deg()(dst)
    for i, (We, bb, Wf, vf, C) in enumerate(
            [(We0, bb0, Wf0, vf0, C0), (We1, bb1, Wf1, vf1, C1)]):
        w1 = p[f'l{i}_em_w1']
        AB = _ab(h, w1[:L], w1[L:2 * L], bb)
        S2 = _segment_pass(AB, C, dst, src)
        h = _node_update(h, S2, deg2d, p[f'l{i}_nm_w1'][:L], Wf,
                         p[f'l{i}_nm_b1'][None, :], vf,
                         p[f'l{i}_nm_w2'], p[f'l{i}_nm_b2'][None, :])

    return _decode(h, p)


# speedup vs baseline: 2.6698x; 1.1413x over previous
import functools

import jax
import jax.numpy as jnp
from jax import lax
from jax.experimental import pallas as pl
from jax.experimental.pallas import tpu as pltpu
from jax.experimental.pallas import tpu_sc as plsc

N = 10000
E = 160000
L = 256
NBLK = 1000
EBLK = 2000
NB = N // NBLK
EB = E // EBLK
F32 = jnp.float32


def _dot(a, b):
    return jax.lax.dot_general(a, b, (((1,), (0,)), ((), ())),
                               preferred_element_type=F32)


def _prep_body(ee_w2, ee_b2, w1e0, w1e1, emb1_0, emb1_1, emw2_0, emw2_1,
               nmw1a0, nmw1a1, emb2_0, emb2_1,
               We0, We1, bb0, bb1, Wf0, Wf1, vf0, vf1):
    We0[...] = _dot(ee_w2[...], w1e0[...])
    We1[...] = _dot(ee_w2[...], w1e1[...])
    bb0[...] = emb1_0[...] + _dot(ee_b2[...], w1e0[...])
    bb1[...] = emb1_1[...] + _dot(ee_b2[...], w1e1[...])
    Wf0[...] = _dot(emw2_0[...], nmw1a0[...])
    Wf1[...] = _dot(emw2_1[...], nmw1a1[...])
    vf0[...] = _dot(emb2_0[...], nmw1a0[...])
    vf1[...] = _dot(emb2_1[...], nmw1a1[...])


def _prep(p):
    mat = jax.ShapeDtypeStruct((L, L), F32)
    vec = jax.ShapeDtypeStruct((1, L), F32)
    return pl.pallas_call(
        _prep_body,
        out_shape=[mat, mat, vec, vec, mat, mat, vec, vec],
    )(p['ee_w2'], p['ee_b2'][None, :],
      p['l0_em_w1'][2 * L:], p['l1_em_w1'][2 * L:],
      p['l0_em_b1'][None, :], p['l1_em_b1'][None, :],
      p['l0_em_w2'], p['l1_em_w2'],
      p['l0_nm_w1'][L:], p['l1_nm_w1'][L:],
      p['l0_em_b2'][None, :], p['l1_em_b2'][None, :])


def _enc_body(x, w1, b1, w2, b2, h):
    t = jnp.maximum(_dot(x[...], w1[...]) + b1[...], 0.0)
    h[...] = _dot(t, w2[...]) + b2[...]


def _node_encode(x, p):
    return pl.pallas_call(
        _enc_body,
        grid=(NB,),
        in_specs=[
            pl.BlockSpec((NBLK, 128), lambda i: (i, 0)),
            pl.BlockSpec((128, L), lambda i: (0, 0)),
            pl.BlockSpec((1, L), lambda i: (0, 0)),
            pl.BlockSpec((L, L), lambda i: (0, 0)),
            pl.BlockSpec((1, L), lambda i: (0, 0)),
        ],
        out_specs=pl.BlockSpec((NBLK, L), lambda i: (i, 0)),
        out_shape=jax.ShapeDtypeStruct((N, L), F32),
    )(x, p['ne_w1'], p['ne_b1'][None, :], p['ne_w2'], p['ne_b2'][None, :])


def _edge_body(ea, w1, b1, We0, We1, C0, C1):
    g = jnp.maximum(_dot(ea[...], w1[...]) + b1[...], 0.0)
    C0[...] = _dot(g, We0[...])
    C1[...] = _dot(g, We1[...])


def _edge_C(edge_attr, p, We0, We1):
    cshape = jax.ShapeDtypeStruct((2 * E, 128), F32)
    return pl.pallas_call(
        _edge_body,
        grid=(2, EB),
        in_specs=[
            pl.BlockSpec((EBLK, 16), lambda h, i: (i, 0)),
            pl.BlockSpec((16, L), lambda h, i: (0, 0)),
            pl.BlockSpec((1, L), lambda h, i: (0, 0)),
            pl.BlockSpec((L, 128), lambda h, i: (0, h)),
            pl.BlockSpec((L, 128), lambda h, i: (0, h)),
        ],
        out_specs=[
            pl.BlockSpec((EBLK, 128), lambda h, i: (h * EB + i, 0)),
            pl.BlockSpec((EBLK, 128), lambda h, i: (h * EB + i, 0)),
        ],
        out_shape=[cshape, cshape],
    )(edge_attr, p['ee_w1'], p['ee_b1'][None, :], We0, We1)


def _ab_body(h, w1d, w1s, bb, A0, A1, B0, B1):
    hv = h[...]
    a = _dot(hv, w1d[...]) + bb[...]
    b = _dot(hv, w1s[...])
    A0[...] = a[:, :128]
    A1[...] = a[:, 128:]
    B0[...] = b[:, :128]
    B1[...] = b[:, 128:]


def _ab(h, w1d, w1s, bb):
    oshape = jax.ShapeDtypeStruct((N, 128), F32)
    return pl.pallas_call(
        _ab_body,
        grid=(NB,),
        in_specs=[
            pl.BlockSpec((NBLK, L), lambda i: (i, 0)),
            pl.BlockSpec((L, L), lambda i: (0, 0)),
            pl.BlockSpec((L, L), lambda i: (0, 0)),
            pl.BlockSpec((1, L), lambda i: (0, 0)),
        ],
        out_specs=[
            pl.BlockSpec((NBLK, 128), lambda i: (i, 0)),
            pl.BlockSpec((NBLK, 128), lambda i: (i, 0)),
            pl.BlockSpec((NBLK, 128), lambda i: (i, 0)),
            pl.BlockSpec((NBLK, 128), lambda i: (i, 0)),
        ],
        out_shape=[oshape, oshape, oshape, oshape],
    )(h, w1d, w1s, bb)


def _upd_body(h, St, Sb, dg, W1h, Wf, b1, vf, w2, b2, hn):
    hv = h[...]
    wf = Wf[...]
    t = (_dot(hv, W1h[...]) + _dot(St[...], wf[:128]) +
         _dot(Sb[...], wf[128:]) + dg[:, 0:1] * vf[...] + b1[...])
    t = jnp.maximum(t, 0.0)
    hn[...] = hv + _dot(t, w2[...]) + b2[...]


def _node_update(h, S2, deg2d, W1h, Wf, b1, vf, w2, b2):
    return pl.pallas_call(
        _upd_body,
        grid=(NB,),
        in_specs=[
            pl.BlockSpec((NBLK, L), lambda i: (i, 0)),
            pl.BlockSpec((NBLK, 128), lambda i: (i, 0)),
            pl.BlockSpec((NBLK, 128), lambda i: (NB + i, 0)),
            pl.BlockSpec((NBLK, 128), lambda i: (i, 0)),
            pl.BlockSpec((L, L), lambda i: (0, 0)),
            pl.BlockSpec((L, L), lambda i: (0, 0)),
            pl.BlockSpec((1, L), lambda i: (0, 0)),
            pl.BlockSpec((1, L), lambda i: (0, 0)),
            pl.BlockSpec((L, L), lambda i: (0, 0)),
            pl.BlockSpec((1, L), lambda i: (0, 0)),
        ],
        out_specs=pl.BlockSpec((NBLK, L), lambda i: (i, 0)),
        out_shape=jax.ShapeDtypeStruct((N, L), F32),
    )(h, S2, S2, deg2d, W1h, Wf, b1, vf, w2, b2)


def _dec_body(h, w1, b1, w2, b2, o):
    t = jnp.maximum(_dot(h[...], w1[...]) + b1[...], 0.0)
    o[...] = _dot(t, w2[...]) + b2[...]


def _decode(h, p):
    w2p = jnp.pad(p['de_w2'], ((0, 0), (0, 128 - 3)))
    b2p = jnp.pad(p['de_b2'], ((0, 128 - 3)))[None, :]
    out = pl.pallas_call(
        _dec_body,
        grid=(NB,),
        in_specs=[
            pl.BlockSpec((NBLK, L), lambda i: (i, 0)),
            pl.BlockSpec((L, L), lambda i: (0, 0)),
            pl.BlockSpec((1, L), lambda i: (0, 0)),
            pl.BlockSpec((L, 128), lambda i: (0, 0)),
            pl.BlockSpec((1, 128), lambda i: (0, 0)),
        ],
        out_specs=pl.BlockSpec((NBLK, 128), lambda i: (i, 0)),
        out_shape=jax.ShapeDtypeStruct((N, 128), F32),
    )(h, p['de_w1'], p['de_b1'][None, :], w2p, b2p)
    return out[:, :3]


CH = 80
EPW = E // 16
NCHK = EPW // CH
NSLAB = 624
NTAIL = N - 16 * NSLAB


def _slabbed(copy_chunk, s):
    for j in range(NSLAB // CH):
        copy_chunk(s * NSLAB + j * CH, CH)
    rem = NSLAB % CH
    if rem:
        copy_chunk(s * NSLAB + (NSLAB // CH) * CH, rem)

    @pl.when(s == 15)
    def _():
        copy_chunk(16 * NSLAB, NTAIL)


def _sc_body(dst, src, A0, A1, B0, B1, C2, S2, s_sh,
             idx_d0, idx_d1, idx_s0, idx_s1, abuf0, abuf1,
             bbuf0, bbuf1, cbuf, sem_ic0, sem_ic1,
             sem_ab0, sem_ab1):
    c = lax.axis_index("c")
    s = lax.axis_index("s")
    cN = c * N
    idx_d = (idx_d0, idx_d1)
    idx_s = (idx_s0, idx_s1)
    abuf = (abuf0, abuf1)
    bbuf = (bbuf0, bbuf1)
    sem_ic = (sem_ic0, sem_ic1)
    sem_ab = (sem_ab0, sem_ab1)

    zero = jnp.zeros((16,), F32)

    def _init_row(r, carry):
        for k2 in range(8):
            abuf0[r, pl.ds(k2 * 16, 16)] = zero
        return carry
    lax.fori_loop(0, CH, _init_row, 0)

    _slabbed(lambda o, n: pltpu.sync_copy(
        abuf0.at[pl.ds(0, n)], s_sh.at[pl.ds(o, n)]), s)
    plsc.subcore_barrier()

    def _start_ic(j, p):
        base = s * EPW + j * CH
        pltpu.async_copy(dst.at[pl.ds(base, CH)], idx_d[p], sem_ic[p])
        pltpu.async_copy(src.at[pl.ds(base, CH)], idx_s[p], sem_ic[p])

    def _wait_ic(p):
        pltpu.make_async_copy(dst.at[pl.ds(0, CH)], idx_d[p], sem_ic[p]).wait()
        pltpu.make_async_copy(src.at[pl.ds(0, CH)], idx_s[p], sem_ic[p]).wait()

    def _start_ab(p):
        @pl.when(c == 0)
        def _():
            pltpu.async_copy(A0.at[idx_d[p]], abuf[p], sem_ab[p])
            pltpu.async_copy(B0.at[idx_s[p]], bbuf[p], sem_ab[p])

        @pl.when(c == 1)
        def _():
            pltpu.async_copy(A1.at[idx_d[p]], abuf[p], sem_ab[p])
            pltpu.async_copy(B1.at[idx_s[p]], bbuf[p], sem_ab[p])

    def _wait_ab(p):
        pltpu.make_async_copy(A0.at[idx_d[p]], abuf[p], sem_ab[p]).wait()
        pltpu.make_async_copy(B0.at[idx_s[p]], bbuf[p], sem_ab[p]).wait()

    def _compute_scatter(j, p):
        base = s * EPW + j * CH
        ab, bb = abuf[p], bbuf[p]
        for half in (0, 1):
            off = half * (CH // 2)
            pltpu.sync_copy(
                C2.at[pl.ds(c * E + base + off, CH // 2)], cbuf)

            def _row(r, rcarry):
                rr = off + r
                for k2 in range(8):
                    sl = pl.ds(k2 * 16, 16)
                    ab[rr, sl] = jnp.maximum(
                        ab[rr, sl] + bb[rr, sl] + cbuf[r, sl], 0.0)
                return rcarry
            lax.fori_loop(0, CH // 2, _row, 0)
        pltpu.sync_copy(ab, s_sh.at[idx_d[p]], add=True)

    _start_ic(0, 0)
    _wait_ic(0)
    _start_ab(0)
    _start_ic(1, 1)

    def _pair(jj, carry):
        for p in (0, 1):
            j = 2 * jj + p

            @pl.when(j < NCHK)
            def _():
                @pl.when(j + 1 < NCHK)
                def _():
                    _wait_ic(1 - p)
                    _start_ab(1 - p)
                _wait_ab(p)
                _compute_scatter(j, p)

                @pl.when(j + 2 < NCHK)
                def _():
                    _start_ic(j + 2, p)
        return carry

    lax.fori_loop(0, (NCHK + 1) // 2, _pair, 0)
    plsc.subcore_barrier()

    def _wb_s(o, n):
        pltpu.sync_copy(s_sh.at[pl.ds(o, n)], abuf0.at[pl.ds(0, n)])
        pltpu.sync_copy(abuf0.at[pl.ds(0, n)], S2.at[pl.ds(cN + o, n)])
    _slabbed(_wb_s, s)


def _make_sc():
    i32 = jnp.int32
    return pl.kernel(
        _sc_body,
        out_type=[jax.ShapeDtypeStruct((2 * N, 128), F32)],
        mesh=plsc.VectorSubcoreMesh(core_axis_name="c", subcore_axis_name="s"),
        scratch_types=[
            pltpu.VMEM_SHARED((N, 128), F32),
            pltpu.VMEM((CH,), i32), pltpu.VMEM((CH,), i32),
            pltpu.VMEM((CH,), i32), pltpu.VMEM((CH,), i32),
            pltpu.VMEM((CH, 128), F32), pltpu.VMEM((CH, 128), F32),
            pltpu.VMEM((CH, 128), F32), pltpu.VMEM((CH, 128), F32),
            pltpu.VMEM((CH // 2, 128), F32),
            pltpu.SemaphoreType.DMA, pltpu.SemaphoreType.DMA,
            pltpu.SemaphoreType.DMA, pltpu.SemaphoreType.DMA,
        ],
    )


def _deg_body(dst, deg2d, d_sh, idx_d, obuf):
    c = lax.axis_index("c")
    s = lax.axis_index("s")

    zero = jnp.zeros((16,), F32)
    one = jnp.full((16,), 1.0, dtype=F32)

    def _fill(val):
        def _row(r, carry):
            for k2 in range(8):
                obuf[r, pl.ds(k2 * 16, 16)] = val
            return carry
        lax.fori_loop(0, CH, _row, 0)

    _fill(zero)

    @pl.when(c == 0)
    def _():
        _slabbed(lambda o, n: pltpu.sync_copy(
            obuf.at[pl.ds(0, n)], d_sh.at[pl.ds(o, n)]), s)
    _fill(one)
    plsc.subcore_barrier()

    @pl.when(c == 0)
    def _():
        def _chunk(j, carry):
            base = s * EPW + j * CH
            pltpu.sync_copy(dst.at[pl.ds(base, CH)], idx_d)
            pltpu.sync_copy(obuf, d_sh.at[idx_d], add=True)
            return carry
        lax.fori_loop(0, NCHK, _chunk, 0)
    plsc.subcore_barrier()

    @pl.when(c == 0)
    def _():
        def _wb_d(o, n):
            pltpu.sync_copy(d_sh.at[pl.ds(o, n)], obuf.at[pl.ds(0, n)])
            pltpu.sync_copy(obuf.at[pl.ds(0, n)], deg2d.at[pl.ds(o, n)])
        _slabbed(_wb_d, s)


def _make_deg():
    return pl.kernel(
        _deg_body,
        out_type=[jax.ShapeDtypeStruct((N, 128), F32)],
        mesh=plsc.VectorSubcoreMesh(core_axis_name="c", subcore_axis_name="s"),
        scratch_types=[
            pltpu.VMEM_SHARED((N, 128), F32),
            pltpu.VMEM((CH,), jnp.int32),
            pltpu.VMEM((CH, 128), F32),
        ],
    )


def _segment_pass(AB, C2, dst, src):
    A0, A1, B0, B1 = AB
    (S2,) = _make_sc()(dst, src, A0, A1, B0, B1, C2)
    return S2


def kernel(x, edge_index, edge_attr, params):
    p = params
    src = edge_index[0]
    dst = edge_index[1]

    We0, We1, bb0, bb1, Wf0, Wf1, vf0, vf1 = _prep(p)
    h = _node_encode(x, p)
    C0, C1 = _edge_C(edge_attr, p, We0, We1)

    (deg2d,) = _make_deg()(dst)
    for i, (We, bb, Wf, vf, C) in enumerate(
            [(We0, bb0, Wf0, vf0, C0), (We1, bb1, Wf1, vf1, C1)]):
        w1 = p[f'l{i}_em_w1']
        AB = _ab(h, w1[:L], w1[L:2 * L], bb)
        S2 = _segment_pass(AB, C, dst, src)
        h = _node_update(h, S2, deg2d, p[f'l{i}_nm_w1'][:L], Wf,
                         p[f'l{i}_nm_b1'][None, :], vf,
                         p[f'l{i}_nm_w2'], p[f'l{i}_nm_b2'][None, :])

    return _decode(h, p)
